# Initial kernel scaffold; baseline (speedup 1.0000x reference)
import functools

import jax
import jax.numpy as jnp
from jax.experimental import pallas as pl
from jax.experimental.pallas import tpu as pltpu

_FEAT = 256
_SCALAR = 30.0
_LANE = 256


def _oim_kernel(f_ref, lab_ref, w_ref, out_ref, acc_ref, cnt_ref, *, n_classes, n_steps):
    step = pl.program_id(0)

    f = f_ref[...]
    w = w_ref[...]
    s = jax.lax.dot_general(
        f, w, (((1,), (0,)), ((), ())), preferred_element_type=jnp.float32
    )
    s = s * _SCALAR

    col = jax.lax.broadcasted_iota(jnp.int32, s.shape, 1)
    lab = lab_ref[...]
    valid = lab > -1
    safe_lab = jnp.where(valid, lab, 0)

    e = jnp.where(col < n_classes, jnp.exp(s - _SCALAR), 0.0)
    se = jnp.sum(e, axis=1)

    ls = jnp.sum(jnp.where(col == safe_lab[:, None], s, 0.0), axis=1)

    nll = (_SCALAR + jnp.log(se)) - ls
    part = jnp.sum(jnp.where(valid, nll, 0.0))
    pcnt = jnp.sum(valid.astype(jnp.float32))

    @pl.when(step == 0)
    def _init():
        acc_ref[0] = part
        cnt_ref[0] = pcnt

    @pl.when(step > 0)
    def _acc():
        acc_ref[0] += part
        cnt_ref[0] += pcnt

    @pl.when(step == n_steps - 1)
    def _fin():
        out_ref[0, 0] = acc_ref[0] / jnp.maximum(cnt_ref[0], 1.0)


@jax.jit
def kernel(features, pid_labels, lookup_table, queue):
    n_rows, feat = features.shape
    n_classes = lookup_table.shape[0] + queue.shape[0]
    np_pad = ((n_classes + _LANE - 1) // _LANE) * _LANE

    w = jnp.concatenate([lookup_table, queue], axis=0)
    w = jnp.pad(w, ((0, np_pad - n_classes), (0, 0)))
    wt = w.T.astype(jnp.bfloat16)
    f16 = features.astype(jnp.bfloat16)

    bm = 256
    n_steps = n_rows // bm

    out = pl.pallas_call(
        functools.partial(_oim_kernel, n_classes=n_classes, n_steps=n_steps),
        grid=(n_steps,),
        in_specs=[
            pl.BlockSpec((bm, feat), lambda i: (i, 0)),
            pl.BlockSpec((bm,), lambda i: (i,)),
            pl.BlockSpec((feat, np_pad), lambda i: (0, 0)),
        ],
        out_specs=pl.BlockSpec((1, 1), lambda i: (0, 0)),
        out_shape=jax.ShapeDtypeStruct((1, 1), jnp.float32),
        scratch_shapes=[
            pltpu.SMEM((1,), jnp.float32),
            pltpu.SMEM((1,), jnp.float32),
        ],
    )(f16, pid_labels, wt)
    return out[0, 0]


# speedup vs baseline: 4.4993x; 4.4993x over previous
import functools

import jax
import jax.numpy as jnp
from jax.experimental import pallas as pl
from jax.experimental.pallas import tpu as pltpu

_FEAT = 256
_SCALAR = 30.0
_LANE = 256


def _oim_kernel(f_ref, lab_ref, w_ref, out_ref, acc_ref, cnt_ref, *, n_classes, n_steps):
    step = pl.program_id(0)

    f = f_ref[...]
    w = w_ref[...]
    s = jax.lax.dot_general(
        f, w, (((1,), (0,)), ((), ())), preferred_element_type=jnp.float32
    )
    s = s * _SCALAR

    col = jax.lax.broadcasted_iota(jnp.int32, s.shape, 1)
    lab = lab_ref[...]
    valid = lab > -1
    safe_lab = jnp.where(valid, lab, 0)

    e = jnp.where(col < n_classes, jnp.exp(s - _SCALAR), 0.0)
    se = jnp.sum(e, axis=1)

    ls = jnp.sum(jnp.where(col == safe_lab[:, None], s, 0.0), axis=1)

    nll = (_SCALAR + jnp.log(se)) - ls
    part = jnp.sum(jnp.where(valid, nll, 0.0))
    pcnt = jnp.sum(valid.astype(jnp.float32))

    @pl.when(step == 0)
    def _init():
        acc_ref[0] = part
        cnt_ref[0] = pcnt

    @pl.when(step > 0)
    def _acc():
        acc_ref[0] += part
        cnt_ref[0] += pcnt

    @pl.when(step == n_steps - 1)
    def _fin():
        out_ref[...] = (acc_ref[0] / jnp.maximum(cnt_ref[0], 1.0)).reshape(1, 1)


@jax.jit
def kernel(features, pid_labels, lookup_table, queue):
    n_rows, feat = features.shape
    n_classes = lookup_table.shape[0] + queue.shape[0]
    np_pad = ((n_classes + _LANE - 1) // _LANE) * _LANE

    w = jnp.concatenate([lookup_table, queue], axis=0)
    w = jnp.pad(w, ((0, np_pad - n_classes), (0, 0)))
    wt = w.T.astype(jnp.bfloat16)
    f16 = features.astype(jnp.bfloat16)

    bm = 256
    n_steps = n_rows // bm

    out = pl.pallas_call(
        functools.partial(_oim_kernel, n_classes=n_classes, n_steps=n_steps),
        grid=(n_steps,),
        in_specs=[
            pl.BlockSpec((bm, feat), lambda i: (i, 0)),
            pl.BlockSpec((bm,), lambda i: (i,)),
            pl.BlockSpec((feat, np_pad), lambda i: (0, 0)),
        ],
        out_specs=pl.BlockSpec((1, 1), lambda i: (0, 0)),
        out_shape=jax.ShapeDtypeStruct((1, 1), jnp.float32),
        scratch_shapes=[
            pltpu.SMEM((1,), jnp.float32),
            pltpu.SMEM((1,), jnp.float32),
        ],
    )(f16, pid_labels, wt)
    return out[0, 0]


# speedup vs baseline: 5.3523x; 1.1896x over previous
import functools

import jax
import jax.numpy as jnp
from jax.experimental import pallas as pl
from jax.experimental.pallas import tpu as pltpu

_FEAT = 256
_SCALAR = 30.0
_LANE = 256


def _oim_kernel(f_ref, lab_ref, w_ref, out_ref, acc_ref, cnt_ref, *, n_classes, n_steps):
    step = pl.program_id(0)

    f = f_ref[...]
    w = w_ref[...]
    s = jax.lax.dot_general(
        f, w, (((1,), (0,)), ((), ())), preferred_element_type=jnp.float32
    )

    col = jax.lax.broadcasted_iota(jnp.int32, s.shape, 1)
    lab = lab_ref[...]
    valid = lab > -1
    safe_lab = jnp.where(valid, lab, 0)

    e = jnp.where(col < n_classes, jnp.exp(s), 0.0)
    se = jnp.sum(e, axis=1)

    ls = jnp.sum(jnp.where(col == safe_lab[:, None], s, 0.0), axis=1)

    nll = jnp.log(se) - ls
    part = jnp.sum(jnp.where(valid, nll, 0.0))
    pcnt = jnp.sum(valid.astype(jnp.float32))

    @pl.when(step == 0)
    def _init():
        acc_ref[0] = part
        cnt_ref[0] = pcnt

    @pl.when(step > 0)
    def _acc():
        acc_ref[0] += part
        cnt_ref[0] += pcnt

    @pl.when(step == n_steps - 1)
    def _fin():
        out_ref[...] = (acc_ref[0] / jnp.maximum(cnt_ref[0], 1.0)).reshape(1, 1)


@jax.jit
def kernel(features, pid_labels, lookup_table, queue):
    n_rows, feat = features.shape
    n_classes = lookup_table.shape[0] + queue.shape[0]
    np_pad = ((n_classes + _LANE - 1) // _LANE) * _LANE

    w = jnp.concatenate([lookup_table, queue], axis=0)
    w = jnp.pad(w, ((0, np_pad - n_classes), (0, 0)))
    wt = (w.T * _SCALAR).astype(jnp.bfloat16)
    f16 = features.astype(jnp.bfloat16)

    bm = 256
    n_steps = n_rows // bm

    out = pl.pallas_call(
        functools.partial(_oim_kernel, n_classes=n_classes, n_steps=n_steps),
        grid=(n_steps,),
        in_specs=[
            pl.BlockSpec((bm, feat), lambda i: (i, 0)),
            pl.BlockSpec((bm,), lambda i: (i,)),
            pl.BlockSpec((feat, np_pad), lambda i: (0, 0)),
        ],
        out_specs=pl.BlockSpec((1, 1), lambda i: (0, 0)),
        out_shape=jax.ShapeDtypeStruct((1, 1), jnp.float32),
        scratch_shapes=[
            pltpu.SMEM((1,), jnp.float32),
            pltpu.SMEM((1,), jnp.float32),
        ],
    )(f16, pid_labels, wt)
    return out[0, 0]


# speedup vs baseline: 7.3067x; 1.3651x over previous
import functools

import jax
import jax.numpy as jnp
from jax.experimental import pallas as pl
from jax.experimental.pallas import tpu as pltpu
from jax.experimental.pallas import tpu_sc as plsc

_SCALAR = 30.0
_LANE = 256
_BM = 256
_BC = 2048
_GW = 128


def _lse_kernel(f_ref, w_ref, out_ref, *, n_classes, np_pad):
    f = f_ref[...]
    w = w_ref[...]
    s = jax.lax.dot_general(
        f, w, (((1,), (0,)), ((), ())), preferred_element_type=jnp.float32
    )
    ncut = (n_classes // _LANE) * _LANE
    se = jnp.sum(jnp.exp(s[:, :ncut]), axis=1)
    if ncut < np_pad:
        tail = s[:, ncut:]
        col = jax.lax.broadcasted_iota(jnp.int32, tail.shape, 1)
        se = se + jnp.sum(
            jnp.where(col < n_classes - ncut, jnp.exp(tail), 0.0), axis=1
        )
    out_ref[...] = jnp.log(se)


def _combine_kernel(f_ref, g_ref, lab_ref, lse_ref, out_ref, acc_ref, cnt_ref,
                    *, n_steps):
    step = pl.program_id(0)
    d = jnp.sum(f_ref[...] * g_ref[...], axis=1)
    lab = lab_ref[...]
    valid = lab > -1
    nll = lse_ref[...] - _SCALAR * d
    part = jnp.sum(jnp.where(valid, nll, 0.0))
    pcnt = jnp.sum(valid.astype(jnp.float32))

    @pl.when(step == 0)
    def _init():
        acc_ref[0] = part
        cnt_ref[0] = pcnt

    @pl.when(step > 0)
    def _acc():
        acc_ref[0] += part
        cnt_ref[0] += pcnt

    @pl.when(step == n_steps - 1)
    def _fin():
        out_ref[...] = (acc_ref[0] / jnp.maximum(cnt_ref[0], 1.0)).reshape(1, 1)


def _sc_gather(table, safe_lab):
    n_rows = safe_lab.shape[0]
    feat = table.shape[1]
    idx2d = safe_lab.reshape(1, n_rows)

    @pl.kernel(
        out_type=jax.ShapeDtypeStruct((n_rows, feat), table.dtype),
        mesh=plsc.VectorSubcoreMesh(core_axis_name="core",
                                    subcore_axis_name="subcore"),
    )
    def gather_kernel(t_hbm, i_hbm, o_hbm):
        def body(i_vmem, o_vmem):
            pltpu.sync_copy(t_hbm.at[i_vmem.at[0]], o_vmem)

        pltpu.emit_pipeline(
            body,
            grid=(n_rows // _GW,),
            in_specs=[pl.BlockSpec((1, _GW), index_map=lambda i: (0, i))],
            out_specs=[pl.BlockSpec((_GW, feat), index_map=lambda i: (i, 0))],
            core_axis_name=("core", "subcore"),
            dimension_semantics=(pltpu.PARALLEL,),
        )(i_hbm, o_hbm)

    return gather_kernel(table, idx2d)


@jax.jit
def kernel(features, pid_labels, lookup_table, queue):
    n_rows, feat = features.shape
    n_classes = lookup_table.shape[0] + queue.shape[0]
    np_pad = ((n_classes + _LANE - 1) // _LANE) * _LANE

    w = jnp.concatenate([lookup_table, queue], axis=0)
    w = jnp.pad(w, ((0, np_pad - n_classes), (0, 0)))
    wt = (w.T * _SCALAR).astype(jnp.bfloat16)
    f16 = features.astype(jnp.bfloat16)

    valid = pid_labels > -1
    safe_lab = jnp.where(valid, pid_labels, 0)

    g = _sc_gather(lookup_table, safe_lab)

    n_steps = n_rows // _BM
    lse = pl.pallas_call(
        functools.partial(_lse_kernel, n_classes=n_classes, np_pad=np_pad),
        grid=(n_steps,),
        in_specs=[
            pl.BlockSpec((_BM, feat), lambda i: (i, 0)),
            pl.BlockSpec((feat, np_pad), lambda i: (0, 0)),
        ],
        out_specs=pl.BlockSpec((_BM,), lambda i: (i,)),
        out_shape=jax.ShapeDtypeStruct((n_rows,), jnp.float32),
    )(f16, wt)

    c_steps = n_rows // _BC
    out = pl.pallas_call(
        functools.partial(_combine_kernel, n_steps=c_steps),
        grid=(c_steps,),
        in_specs=[
            pl.BlockSpec((_BC, feat), lambda i: (i, 0)),
            pl.BlockSpec((_BC, feat), lambda i: (i, 0)),
            pl.BlockSpec((_BC,), lambda i: (i,)),
            pl.BlockSpec((_BC,), lambda i: (i,)),
        ],
        out_specs=pl.BlockSpec((1, 1), lambda i: (0, 0)),
        out_shape=jax.ShapeDtypeStruct((1, 1), jnp.float32),
        scratch_shapes=[
            pltpu.SMEM((1,), jnp.float32),
            pltpu.SMEM((1,), jnp.float32),
        ],
    )(features, g, pid_labels, lse)
    return out[0, 0]


# speedup vs baseline: 7.7472x; 1.0603x over previous
import functools

import jax
import jax.numpy as jnp
from jax.experimental import pallas as pl
from jax.experimental.pallas import tpu as pltpu
from jax.experimental.pallas import tpu_sc as plsc

_SCALAR = 30.0
_LANE = 256
_BM = 256
_BC = 2048
_GW = 128


def _lse_kernel(f_ref, w_ref, out_ref, *, n_classes, np_pad):
    f = f_ref[...].astype(jnp.bfloat16)
    w = w_ref[...]
    s = jax.lax.dot_general(
        f, w, (((1,), (0,)), ((), ())), preferred_element_type=jnp.float32
    )
    ncut = (n_classes // _LANE) * _LANE
    se = jnp.sum(jnp.exp(s[:, :ncut]), axis=1)
    if ncut < np_pad:
        tail = s[:, ncut:]
        col = jax.lax.broadcasted_iota(jnp.int32, tail.shape, 1)
        se = se + jnp.sum(
            jnp.where(col < n_classes - ncut, jnp.exp(tail), 0.0), axis=1
        )
    out_ref[...] = jnp.log(se)


def _combine_kernel(f_ref, g_ref, lab_ref, lse_ref, out_ref, acc_ref, cnt_ref,
                    *, n_steps):
    step = pl.program_id(0)
    lab = lab_ref[...]
    valid = lab > -1
    d = jnp.sum(f_ref[...] * g_ref[...], axis=1)
    nll = lse_ref[...] - _SCALAR * d
    part = jnp.sum(jnp.where(valid, nll, 0.0))
    pcnt = jnp.sum(valid.astype(jnp.float32))

    @pl.when(step == 0)
    def _init():
        acc_ref[0] = part
        cnt_ref[0] = pcnt

    @pl.when(step > 0)
    def _acc():
        acc_ref[0] += part
        cnt_ref[0] += pcnt

    @pl.when(step == n_steps - 1)
    def _fin():
        out_ref[...] = (acc_ref[0] / jnp.maximum(cnt_ref[0], 1.0)).reshape(1, 1)


def _sc_gather(table, safe_lab):
    n_rows = safe_lab.shape[0]
    feat = table.shape[1]
    idx2d = safe_lab.reshape(1, n_rows)

    @pl.kernel(
        out_type=jax.ShapeDtypeStruct((n_rows, feat), table.dtype),
        mesh=plsc.VectorSubcoreMesh(core_axis_name="core",
                                    subcore_axis_name="subcore"),
    )
    def gather_kernel(t_hbm, i_hbm, o_hbm):
        def body(i_vmem, o_vmem):
            pltpu.sync_copy(t_hbm.at[i_vmem.at[0]], o_vmem)

        pltpu.emit_pipeline(
            body,
            grid=(n_rows // _GW,),
            in_specs=[pl.BlockSpec((1, _GW), index_map=lambda i: (0, i))],
            out_specs=[pl.BlockSpec((_GW, feat), index_map=lambda i: (i, 0))],
            core_axis_name=("core", "subcore"),
            dimension_semantics=(pltpu.PARALLEL,),
        )(i_hbm, o_hbm)

    return gather_kernel(table, idx2d)


@jax.jit
def kernel(features, pid_labels, lookup_table, queue):
    n_rows, feat = features.shape
    n_classes = lookup_table.shape[0] + queue.shape[0]
    np_pad = ((n_classes + _LANE - 1) // _LANE) * _LANE

    w = jnp.concatenate([lookup_table, queue], axis=0)
    w = jnp.pad(w, ((0, np_pad - n_classes), (0, 0)))
    wt = (w.T * _SCALAR).astype(jnp.bfloat16)

    valid = pid_labels > -1
    safe_lab = jnp.where(valid, pid_labels, 0)

    g = _sc_gather(lookup_table, safe_lab)

    n_steps = n_rows // _BM
    lse = pl.pallas_call(
        functools.partial(_lse_kernel, n_classes=n_classes, np_pad=np_pad),
        grid=(n_steps,),
        in_specs=[
            pl.BlockSpec((_BM, feat), lambda i: (i, 0)),
            pl.BlockSpec((feat, np_pad), lambda i: (0, 0)),
        ],
        out_specs=pl.BlockSpec((_BM,), lambda i: (i,)),
        out_shape=jax.ShapeDtypeStruct((n_rows,), jnp.float32),
    )(features, wt)

    c_steps = n_rows // _BC
    out = pl.pallas_call(
        functools.partial(_combine_kernel, n_steps=c_steps),
        grid=(c_steps,),
        in_specs=[
            pl.BlockSpec((_BC, feat), lambda i: (i, 0)),
            pl.BlockSpec((_BC, feat), lambda i: (i, 0)),
            pl.BlockSpec((_BC,), lambda i: (i,)),
            pl.BlockSpec((_BC,), lambda i: (i,)),
        ],
        out_specs=pl.BlockSpec((1, 1), lambda i: (0, 0)),
        out_shape=jax.ShapeDtypeStruct((1, 1), jnp.float32),
        scratch_shapes=[
            pltpu.SMEM((1,), jnp.float32),
            pltpu.SMEM((1,), jnp.float32),
        ],
    )(features, g, pid_labels, lse)
    return out[0, 0]


# speedup vs baseline: 7.9741x; 1.0293x over previous
import functools

import jax
import jax.numpy as jnp
from jax.experimental import pallas as pl
from jax.experimental.pallas import tpu as pltpu
from jax.experimental.pallas import tpu_sc as plsc

_SCALAR = 30.0
_LANE = 256
_BM = 512
_BC = 2048
_GW = 128


def _lse_kernel(f_ref, w_ref, out_ref, *, n_classes, np_pad):
    f = f_ref[...].astype(jnp.bfloat16)
    w = w_ref[...]
    s = jax.lax.dot_general(
        f, w, (((1,), (0,)), ((), ())), preferred_element_type=jnp.float32
    )
    ncut = (n_classes // _LANE) * _LANE
    se = jnp.sum(jnp.exp(s[:, :ncut]), axis=1)
    if ncut < np_pad:
        tail = s[:, ncut:]
        col = jax.lax.broadcasted_iota(jnp.int32, tail.shape, 1)
        se = se + jnp.sum(
            jnp.where(col < n_classes - ncut, jnp.exp(tail), 0.0), axis=1
        )
    out_ref[...] = jnp.log(se)


def _combine_kernel(f_ref, g_ref, lab_ref, lse_ref, out_ref, acc_ref, cnt_ref,
                    *, n_steps):
    step = pl.program_id(0)
    lab = lab_ref[...]
    valid = lab > -1
    d = jnp.sum(f_ref[...] * g_ref[...], axis=1)
    nll = lse_ref[...] - _SCALAR * d
    part = jnp.sum(jnp.where(valid, nll, 0.0))
    pcnt = jnp.sum(valid.astype(jnp.float32))

    @pl.when(step == 0)
    def _init():
        acc_ref[0] = part
        cnt_ref[0] = pcnt

    @pl.when(step > 0)
    def _acc():
        acc_ref[0] += part
        cnt_ref[0] += pcnt

    @pl.when(step == n_steps - 1)
    def _fin():
        out_ref[...] = (acc_ref[0] / jnp.maximum(cnt_ref[0], 1.0)).reshape(1, 1)


def _sc_gather(table, safe_lab):
    n_rows = safe_lab.shape[0]
    feat = table.shape[1]
    idx2d = safe_lab.reshape(1, n_rows)

    @pl.kernel(
        out_type=jax.ShapeDtypeStruct((n_rows, feat), table.dtype),
        mesh=plsc.VectorSubcoreMesh(core_axis_name="core",
                                    subcore_axis_name="subcore"),
    )
    def gather_kernel(t_hbm, i_hbm, o_hbm):
        def body(i_vmem, o_vmem):
            pltpu.sync_copy(t_hbm.at[i_vmem.at[0]], o_vmem)

        pltpu.emit_pipeline(
            body,
            grid=(n_rows // _GW,),
            in_specs=[pl.BlockSpec((1, _GW), index_map=lambda i: (0, i))],
            out_specs=[pl.BlockSpec((_GW, feat), index_map=lambda i: (i, 0))],
            core_axis_name=("core", "subcore"),
            dimension_semantics=(pltpu.PARALLEL,),
        )(i_hbm, o_hbm)

    return gather_kernel(table, idx2d)


@jax.jit
def kernel(features, pid_labels, lookup_table, queue):
    n_rows, feat = features.shape
    n_classes = lookup_table.shape[0] + queue.shape[0]
    np_pad = ((n_classes + _LANE - 1) // _LANE) * _LANE

    w = jnp.concatenate([lookup_table, queue], axis=0)
    w = jnp.pad(w, ((0, np_pad - n_classes), (0, 0)))
    wt = (w.T * _SCALAR).astype(jnp.bfloat16)

    valid = pid_labels > -1
    safe_lab = jnp.where(valid, pid_labels, 0)

    g = _sc_gather(lookup_table, safe_lab)

    n_steps = n_rows // _BM
    lse = pl.pallas_call(
        functools.partial(_lse_kernel, n_classes=n_classes, np_pad=np_pad),
        grid=(n_steps,),
        in_specs=[
            pl.BlockSpec((_BM, feat), lambda i: (i, 0)),
            pl.BlockSpec((feat, np_pad), lambda i: (0, 0)),
        ],
        out_specs=pl.BlockSpec((_BM,), lambda i: (i,)),
        out_shape=jax.ShapeDtypeStruct((n_rows,), jnp.float32),
    )(features, wt)

    c_steps = n_rows // _BC
    out = pl.pallas_call(
        functools.partial(_combine_kernel, n_steps=c_steps),
        grid=(c_steps,),
        in_specs=[
            pl.BlockSpec((_BC, feat), lambda i: (i, 0)),
            pl.BlockSpec((_BC, feat), lambda i: (i, 0)),
            pl.BlockSpec((_BC,), lambda i: (i,)),
            pl.BlockSpec((_BC,), lambda i: (i,)),
        ],
        out_specs=pl.BlockSpec((1, 1), lambda i: (0, 0)),
        out_shape=jax.ShapeDtypeStruct((1, 1), jnp.float32),
        scratch_shapes=[
            pltpu.SMEM((1,), jnp.float32),
            pltpu.SMEM((1,), jnp.float32),
        ],
    )(features, g, pid_labels, lse)
    return out[0, 0]


# speedup vs baseline: 8.8559x; 1.1106x over previous
import dataclasses
import functools

import jax
import jax.numpy as jnp
from jax.experimental import pallas as pl
from jax.experimental.pallas import tpu as pltpu
from jax.experimental.pallas import tpu_sc as plsc

_SCALAR = 30.0
_LANE = 256
_BM = 512
_BC = 2048
_GW = 128


def _lse_kernel(f_ref, w_ref, out_ref, *, n_classes, np_pad):
    f = f_ref[...].astype(jnp.bfloat16)
    w = w_ref[...]
    s = jax.lax.dot_general(
        f, w, (((1,), (0,)), ((), ())), preferred_element_type=jnp.float32
    )
    ncut = (n_classes // _LANE) * _LANE
    se = jnp.sum(jnp.exp(s[:, :ncut]), axis=1)
    if ncut < np_pad:
        tail = s[:, ncut:]
        col = jax.lax.broadcasted_iota(jnp.int32, tail.shape, 1)
        se = se + jnp.sum(
            jnp.where(col < n_classes - ncut, jnp.exp(tail), 0.0), axis=1
        )
    out_ref[...] = jnp.log(se)


def _combine_kernel(dp_ref, lab_ref, lse_ref, out_ref):
    lab = lab_ref[...]
    valid = lab > -1
    d = dp_ref[0, :]
    nll = lse_ref[...] - _SCALAR * d
    part = jnp.sum(jnp.where(valid, nll, 0.0))
    pcnt = jnp.sum(valid.astype(jnp.float32))
    out_ref[...] = (part / jnp.maximum(pcnt, 1.0)).reshape(1, 1)


def _sc_gather_dot(table, safe_lab, features):
    n_rows = safe_lab.shape[0]
    feat = table.shape[1]
    nl = 16
    idx2d = safe_lab.reshape(1, n_rows)

    cp = pltpu.CompilerParams()
    if "needs_layout_passes" in pltpu.CompilerParams.__dataclass_fields__:
        cp = dataclasses.replace(cp, needs_layout_passes=False)

    @pl.kernel(
        out_type=jax.ShapeDtypeStruct((1, n_rows), jnp.float32),
        mesh=plsc.VectorSubcoreMesh(core_axis_name="core",
                                    subcore_axis_name="subcore"),
        scratch_types=[pltpu.VMEM((_GW, feat), jnp.float32)],
        compiler_params=cp,
    )
    def gather_kernel(t_hbm, i_hbm, f_hbm, o_hbm, g_scr):
        def body(i_vmem, f_vmem, o_vmem):
            pltpu.sync_copy(t_hbm.at[i_vmem.at[0]], g_scr)

            @pl.loop(0, _GW, step=nl)
            def _(g):
                def row_step(j, vec):
                    def k_step(k, acc):
                        ks = pl.ds(k * nl, nl)
                        return acc + f_vmem[g + j, ks] * g_scr[g + j, ks]

                    acc = jax.lax.fori_loop(
                        0, feat // nl, k_step, jnp.zeros((nl,), jnp.float32)
                    )
                    lane = jax.lax.iota(jnp.int32, nl)
                    return jnp.where(lane == j, jnp.sum(acc), vec)

                o_vmem[0, pl.ds(g, nl)] = jax.lax.fori_loop(
                    0, nl, row_step, jnp.zeros((nl,), jnp.float32)
                )

        pltpu.emit_pipeline(
            body,
            grid=(n_rows // _GW,),
            in_specs=[
                pl.BlockSpec((1, _GW), index_map=lambda i: (0, i)),
                pl.BlockSpec((_GW, feat), index_map=lambda i: (i, 0)),
            ],
            out_specs=[pl.BlockSpec((1, _GW), index_map=lambda i: (0, i))],
            core_axis_name=("core", "subcore"),
            dimension_semantics=(pltpu.PARALLEL,),
        )(i_hbm, f_hbm, o_hbm)

    return gather_kernel(table, idx2d, features)


@jax.jit
def kernel(features, pid_labels, lookup_table, queue):
    n_rows, feat = features.shape
    n_classes = lookup_table.shape[0] + queue.shape[0]
    np_pad = ((n_classes + _LANE - 1) // _LANE) * _LANE

    w = jnp.concatenate([lookup_table, queue], axis=0)
    w = jnp.pad(w, ((0, np_pad - n_classes), (0, 0)))
    wt = (w.T * _SCALAR).astype(jnp.bfloat16)

    valid = pid_labels > -1
    safe_lab = jnp.where(valid, pid_labels, 0)

    dp = _sc_gather_dot(lookup_table, safe_lab, features)

    n_steps = n_rows // _BM
    lse = pl.pallas_call(
        functools.partial(_lse_kernel, n_classes=n_classes, np_pad=np_pad),
        grid=(n_steps,),
        in_specs=[
            pl.BlockSpec((_BM, feat), lambda i: (i, 0)),
            pl.BlockSpec((feat, np_pad), lambda i: (0, 0)),
        ],
        out_specs=pl.BlockSpec((_BM,), lambda i: (i,)),
        out_shape=jax.ShapeDtypeStruct((n_rows,), jnp.float32),
    )(features, wt)

    out = pl.pallas_call(
        _combine_kernel,
        grid=(1,),
        in_specs=[
            pl.BlockSpec((1, n_rows), lambda i: (0, 0)),
            pl.BlockSpec((n_rows,), lambda i: (0,)),
            pl.BlockSpec((n_rows,), lambda i: (0,)),
        ],
        out_specs=pl.BlockSpec((1, 1), lambda i: (0, 0)),
        out_shape=jax.ShapeDtypeStruct((1, 1), jnp.float32),
    )(dp, pid_labels, lse)
    return out[0, 0]
